# baseline (speedup 1.0000x reference)
import functools

import jax
import jax.numpy as jnp
from jax import lax
from jax.experimental import pallas as pl
from jax.experimental.pallas import tpu as pltpu
from jax.experimental.pallas import tpu_sc as plsc

BATCH = 16384
D = 32
L = 16
NC = 2
NS = 16
NW = NC * NS

NIDX = BATCH // L
COLS_W = 244
CHUNK_COLS = 4
CW = CHUNK_COLS * 128
NCHUNK = COLS_W // CHUNK_COLS
RANGE_W = COLS_W * 128
TAIL_LO = NW * RANGE_W
SCAT_RING = 8


def _gather_body(uidx_hbm, iidx_hbm, utab_hbm, itab_hbm, utail_hbm, itail_hbm,
                 ubuf_hbm, ibuf_hbm,
                 idx_v, hit_v, ch_v, buf0, buf1, tail2_v, stage_v,
                 sem0, sem1, sem_sc):
    wid = lax.axis_index("s") * NC + lax.axis_index("c")
    is_last = wid == NW - 1
    rlo = wid * RANGE_W
    rhi = jnp.where(is_last, 1000000, rlo + RANGE_W)
    iota16 = lax.iota(jnp.int32, L)
    dump_vec = jnp.full((L,), BATCH, jnp.int32)

    def drain_one(obuf):
        pltpu.make_async_copy(
            stage_v.at[pl.ds(0, L), :], obuf.at[dump_vec], sem_sc).wait()

    def run_table(idx_hbm, tab, tail_hbm, obuf):
        pltpu.sync_copy(idx_hbm, idx_v)

        def det_body(k, cnt):
            v = idx_v[pl.ds(k * L, L)]
            m = (v >= rlo) & (v < rhi)
            pref = plsc.cumsum(m.astype(jnp.int32))
            pos = pref + (cnt - 1)
            packed = (iota16 + k * L) * 32768 + (v - rlo)
            plsc.store_scatter(hit_v, [pos], packed, mask=m)
            return cnt + pref[L - 1]

        cnt = lax.fori_loop(0, NIDX, det_body, jnp.int32(0))
        nwin = (cnt + L - 1) // L

        def mk_compact(clo_rel, chi_rel):
            def cbody(h, nf):
                p = hit_v[pl.ds(h * L, L)]
                rel = lax.rem(p, 32768)
                m = (rel >= clo_rel) & (rel < chi_rel)
                npos = plsc.all_reduce_population_count(m)[0]

                def keep():
                    ch_v[pl.ds(nf * L, L)] = p

                pl.when(npos > 0)(keep)
                return nf + jnp.where(npos > 0, 1, 0)

            return cbody

        def mk_extract(buf, obuf, clo_rel, chi_rel):
            def ebody(t, nsc):
                def ring_wait():
                    drain_one(obuf)

                pl.when(nsc >= SCAT_RING)(ring_wait)
                p = ch_v[pl.ds(t * L, L)]
                rel = lax.rem(p, 32768)
                m = (rel >= clo_rel) & (rel < chi_rel)
                rl = jnp.where(m, rel - clo_rel, 0)
                j = jnp.where(m, p // 32768, BATCH)
                slot = (nsc % SCAT_RING) * L
                srow = iota16 + slot
                for s in range(D):
                    g = plsc.load_gather(
                        buf, [jnp.full((L,), s, jnp.int32), rl], mask=m)
                    plsc.store_scatter(
                        stage_v, [srow, jnp.full((L,), s, jnp.int32)], g)
                pltpu.async_copy(
                    stage_v.at[pl.ds(slot, L), :], obuf.at[j], sem_sc)
                return nsc + 1

            return ebody

        def chunk_match(buf, clo_rel, chi_rel, nsc):
            nf = lax.fori_loop(0, nwin, mk_compact(clo_rel, chi_rel),
                               jnp.int32(0))
            nsc = lax.fori_loop(
                0, nf, mk_extract(buf, obuf, clo_rel, chi_rel), nsc)
            return nsc

        def fire(ci, buf, sem):
            off = wid * RANGE_W + ci * CW
            pltpu.async_copy(
                tab.at[pl.ds(0, D), pl.ds(pl.multiple_of(off, 128), CW)],
                buf, sem)

        def wait_chunk(buf, sem):
            pltpu.make_async_copy(
                tab.at[pl.ds(0, D), pl.ds(pl.multiple_of(0, 128), CW)],
                buf, sem).wait()

        nsc = jnp.int32(0)
        fire(0, buf0, sem0)
        fire(1, buf1, sem1)

        def pair_body(k, nsc):
            c0 = 2 * k
            wait_chunk(buf0, sem0)
            nsc = chunk_match(buf0, c0 * CW, c0 * CW + CW, nsc)
            fire(c0 + 2, buf0, sem0)
            wait_chunk(buf1, sem1)
            nsc = chunk_match(buf1, (c0 + 1) * CW, (c0 + 1) * CW + CW, nsc)
            pl.when(k < NCHUNK // 2 - 1)(
                lambda: fire(c0 + 3, buf1, sem1))
            return nsc

        nsc = lax.fori_loop(0, NCHUNK // 2, pair_body, nsc)
        wait_chunk(buf0, sem0)
        nsc = chunk_match(buf0, (NCHUNK - 1) * CW, NCHUNK * CW, nsc)

        def tail_fetch():
            pltpu.sync_copy(
                tab.at[pl.ds(0, D), pl.ds(pl.multiple_of(TAIL_LO, 128), CW)],
                buf0)
            pltpu.sync_copy(tail_hbm, tail2_v)

        pl.when(is_last)(tail_fetch)
        nsc = chunk_match(buf0, NCHUNK * CW, NCHUNK * CW + CW, nsc)
        nsc = chunk_match(tail2_v, NCHUNK * CW + CW, NCHUNK * CW + CW + 64, nsc)

        def drain_body(_, c):
            drain_one(obuf)
            return c

        lax.fori_loop(0, jnp.minimum(nsc, SCAT_RING), drain_body, 0)

    run_table(uidx_hbm, utab_hbm, utail_hbm, ubuf_hbm)
    run_table(iidx_hbm, itab_hbm, itail_hbm, ibuf_hbm)


def _finish_body(ubuf_hbm, ibuf_hbm, w_hbm, b_hbm, out_hbm,
                 uc0, uc1, ic0, ic1, w_v, b_v, out_v, semu0, semu1, semi0, semi1):
    wid = lax.axis_index("s") * NC + lax.axis_index("c")
    base = wid * (BATCH // NW)
    iota16 = lax.iota(jnp.int32, L)

    pltpu.sync_copy(w_hbm, w_v)
    pltpu.sync_copy(b_hbm, b_v)
    w_lo = w_v[pl.ds(0, L)]
    w_hi = w_v[pl.ds(L, L)]
    bval = b_v[pl.ds(0, L)][0]

    ucs = (uc0, uc1)
    ics = (ic0, ic1)
    usems = (semu0, semu1)
    isems = (semi0, semi1)

    def fire(ci, slot):
        row0 = base + ci * 128
        cu = pltpu.async_copy(
            ubuf_hbm.at[pl.ds(pl.multiple_of(row0, 128), 128), :],
            ucs[slot], usems[slot])
        cv = pltpu.async_copy(
            ibuf_hbm.at[pl.ds(pl.multiple_of(row0, 128), 128), :],
            ics[slot], isems[slot])
        return cu, cv

    cur = fire(0, 0)
    for ci in range(4):
        nxt = fire(ci + 1, (ci + 1) % 2) if ci + 1 < 4 else None
        cur[0].wait()
        cur[1].wait()
        ub = ucs[ci % 2]
        ib = ics[ci % 2]

        def group_body(g, carry):
            rows = iota16 + g * L
            acc = jnp.zeros((L,), jnp.float32)
            for d in range(D):
                cols = jnp.full((L,), d, jnp.int32)
                u = plsc.load_gather(ub, [rows, cols])
                it = plsc.load_gather(ib, [rows, cols])
                wd = (w_lo if d < L else w_hi)[d % L]
                acc = acc + u * it * wd
            rating = 1.0 / (1.0 + jnp.exp(-(acc + bval)))
            out_v[pl.ds(ci * 128 + g * L, L)] = rating
            return carry

        lax.fori_loop(0, 8, group_body, 0)
        cur = nxt
    pltpu.sync_copy(out_v, out_hbm.at[pl.ds(base, BATCH // NW)])


@jax.jit
def _gmf_sc(uidx, iidx, utab_t, itab_t, w_flat, b_pad):
    mesh = plsc.VectorSubcoreMesh(core_axis_name="c", subcore_axis_name="s")
    gather = functools.partial(
        pl.kernel,
        mesh=mesh,
        compiler_params=pltpu.CompilerParams(needs_layout_passes=False),
        out_type=(jax.ShapeDtypeStruct((BATCH + 1, 128), jnp.float32),
                  jax.ShapeDtypeStruct((BATCH + 1, 128), jnp.float32)),
        scratch_types=[
            pltpu.VMEM((BATCH,), jnp.int32),
            pltpu.VMEM((BATCH,), jnp.int32),
            pltpu.VMEM((BATCH,), jnp.int32),
            pltpu.VMEM((D, CW), jnp.float32),
            pltpu.VMEM((D, CW), jnp.float32),
            pltpu.VMEM((D, 128), jnp.float32),
            pltpu.VMEM((SCAT_RING * L, 128), jnp.float32),
            pltpu.SemaphoreType.DMA,
            pltpu.SemaphoreType.DMA,
            pltpu.SemaphoreType.DMA,
        ],
    )(_gather_body)
    utail = jnp.pad(utab_t[:, TAIL_LO + CW:], ((0, 0), (0, 64)))
    itail = jnp.pad(itab_t[:, TAIL_LO + CW:], ((0, 0), (0, 64)))
    ubuf, ibuf = gather(uidx, iidx, utab_t, itab_t, utail, itail)

    finish = functools.partial(
        pl.kernel,
        mesh=mesh,
        compiler_params=pltpu.CompilerParams(needs_layout_passes=False),
        out_type=jax.ShapeDtypeStruct((BATCH,), jnp.float32),
        scratch_types=[
            pltpu.VMEM((128, 128), jnp.float32),
            pltpu.VMEM((128, 128), jnp.float32),
            pltpu.VMEM((128, 128), jnp.float32),
            pltpu.VMEM((128, 128), jnp.float32),
            pltpu.VMEM((D,), jnp.float32),
            pltpu.VMEM((L,), jnp.float32),
            pltpu.VMEM((BATCH // NW,), jnp.float32),
            pltpu.SemaphoreType.DMA,
            pltpu.SemaphoreType.DMA,
            pltpu.SemaphoreType.DMA,
            pltpu.SemaphoreType.DMA,
        ],
    )(_finish_body)
    return finish(ubuf, ibuf, w_flat, b_pad)


def kernel(user_indices, item_indices, user_table, item_table, W, b):
    w_flat = W.reshape(D)
    b_pad = jnp.pad(b.astype(jnp.float32), (0, L - b.shape[0]))
    out = _gmf_sc(user_indices.astype(jnp.int32), item_indices.astype(jnp.int32),
                  user_table.T, item_table.T, w_flat, b_pad)
    return out.reshape(BATCH, 1)


# speedup vs baseline: 18.0390x; 18.0390x over previous
import functools

import jax
import jax.numpy as jnp
from jax import lax
from jax.experimental import pallas as pl
from jax.experimental.pallas import tpu as pltpu
from jax.experimental.pallas import tpu_sc as plsc

BATCH = 16384
D = 32
L = 16
NC = 2
NS = 16
NW = NC * NS
BPW = BATCH // NW
NP = 2
RPP = BPW // NP
NGP = RPP // L
NCONT = 250000


def _sc_body(uidx_hbm, iidx_hbm, utab_hbm, itab_hbm, w_hbm, b_hbm, out_hbm,
             uidx_v, iidx_v, uc_v, ic_v, urows_v, irows_v, w_v, b_v, out_v,
             sem_u, sem_i):
    wid = lax.axis_index("s") * NC + lax.axis_index("c")
    base = wid * BPW

    pltpu.sync_copy(uidx_hbm.at[pl.ds(base, BPW)], uidx_v)
    pltpu.sync_copy(iidx_hbm.at[pl.ds(base, BPW)], iidx_v)
    pltpu.sync_copy(w_hbm, w_v)
    pltpu.sync_copy(b_hbm, b_v)

    lanes = lax.iota(jnp.int32, L)
    bval = b_v[pl.ds(0, L)][0]
    w_lo = w_v[pl.ds(0, L)]
    w_hi = w_v[pl.ds(L, L)]

    for p in range(NP):
        for g in range(NGP):
            off = p * RPP + g * L
            uc_v[pl.ds(g * L, L)] = lax.shift_right_logical(
                uidx_v[pl.ds(off, L)], 2)
            ic_v[pl.ds(g * L, L)] = lax.shift_right_logical(
                iidx_v[pl.ds(off, L)], 2)

        cp_u = pltpu.async_copy(utab_hbm.at[uc_v], urows_v, sem_u)
        cp_i = pltpu.async_copy(itab_hbm.at[ic_v], irows_v, sem_i)
        cp_u.wait()
        cp_i.wait()

        def group_body(g, carry):
            off = p * RPP + g * L
            rows = lanes + g * L
            uv = uidx_v[pl.ds(off, L)]
            iv = iidx_v[pl.ds(off, L)]
            offu = (uv & 3) * 32
            offi = (iv & 3) * 32
            acc = jnp.zeros((L,), jnp.float32)
            for d in range(D):
                u = plsc.load_gather(urows_v, [rows, offu + d])
                it = plsc.load_gather(irows_v, [rows, offi + d])
                wd = (w_lo if d < L else w_hi)[d % L]
                acc = acc + u * it * wd
            rating = 1.0 / (1.0 + jnp.exp(-(acc + bval)))
            out_v[pl.ds(off, L)] = rating
            return carry

        lax.fori_loop(0, NGP, group_body, 0)

    pltpu.sync_copy(out_v, out_hbm.at[pl.ds(base, BPW)])


@jax.jit
def _gmf_sc(uidx, iidx, utab_c, itab_c, w_flat, b_pad):
    mesh = plsc.VectorSubcoreMesh(core_axis_name="c", subcore_axis_name="s")
    f = functools.partial(
        pl.kernel,
        mesh=mesh,
        compiler_params=pltpu.CompilerParams(needs_layout_passes=False),
        out_type=jax.ShapeDtypeStruct((BATCH,), jnp.float32),
        scratch_types=[
            pltpu.VMEM((BPW,), jnp.int32),
            pltpu.VMEM((BPW,), jnp.int32),
            pltpu.VMEM((RPP,), jnp.int32),
            pltpu.VMEM((RPP,), jnp.int32),
            pltpu.VMEM((RPP, 128), jnp.float32),
            pltpu.VMEM((RPP, 128), jnp.float32),
            pltpu.VMEM((D,), jnp.float32),
            pltpu.VMEM((L,), jnp.float32),
            pltpu.VMEM((BPW,), jnp.float32),
            pltpu.SemaphoreType.DMA,
            pltpu.SemaphoreType.DMA,
        ],
    )(_sc_body)
    return f(uidx, iidx, utab_c, itab_c, w_flat, b_pad)


def kernel(user_indices, item_indices, user_table, item_table, W, b):
    w_flat = W.reshape(D)
    b_pad = jnp.pad(b.astype(jnp.float32), (0, L - b.shape[0]))
    utab_c = user_table.reshape(NCONT, 128)
    itab_c = item_table.reshape(NCONT, 128)
    out = _gmf_sc(user_indices.astype(jnp.int32), item_indices.astype(jnp.int32),
                  utab_c, itab_c, w_flat, b_pad)
    return out.reshape(BATCH, 1)


# speedup vs baseline: 18.2245x; 1.0103x over previous
import functools

import jax
import jax.numpy as jnp
from jax import lax
from jax.experimental import pallas as pl
from jax.experimental.pallas import tpu as pltpu
from jax.experimental.pallas import tpu_sc as plsc

BATCH = 16384
D = 32
L = 16
NC = 2
NS = 16
NW = NC * NS
BPW = BATCH // NW
NG = BPW // L


def _sc_body(uidx_hbm, iidx_hbm, utab_hbm, itab_hbm, w_hbm, b_hbm, out_hbm,
             uidx_v, iidx_v, urows_v, irows_v, w_v, b_v, out_v, sem_u, sem_i):
    wid = lax.axis_index("s") * NC + lax.axis_index("c")
    base = wid * BPW

    pltpu.sync_copy(uidx_hbm.at[pl.ds(base, BPW)], uidx_v)
    pltpu.sync_copy(iidx_hbm.at[pl.ds(base, BPW)], iidx_v)

    cp_u = pltpu.async_copy(utab_hbm.at[uidx_v], urows_v, sem_u)
    cp_i = pltpu.async_copy(itab_hbm.at[iidx_v], irows_v, sem_i)
    pltpu.sync_copy(w_hbm, w_v)
    pltpu.sync_copy(b_hbm, b_v)
    cp_u.wait()
    cp_i.wait()

    lanes = lax.iota(jnp.int32, L)
    bval = b_v[pl.ds(0, L)][0]
    w_lo = w_v[pl.ds(0, L)]
    w_hi = w_v[pl.ds(L, L)]

    def group_body(g, carry):
        rows = lanes + g * L
        acc = jnp.zeros((L,), jnp.float32)
        for d in range(D):
            cols = jnp.full((L,), d, jnp.int32)
            u = plsc.load_gather(urows_v, [rows, cols])
            it = plsc.load_gather(irows_v, [rows, cols])
            wd = (w_lo if d < L else w_hi)[d % L]
            acc = acc + u * it * wd
        logits = acc + bval
        rating = 1.0 / (1.0 + jnp.exp(-logits))
        out_v[pl.ds(g * L, L)] = rating
        return carry

    lax.fori_loop(0, NG, group_body, 0)
    pltpu.sync_copy(out_v, out_hbm.at[pl.ds(base, BPW)])


@jax.jit
def _gmf_sc(uidx, iidx, utab, itab, w_flat, b_pad):
    mesh = plsc.VectorSubcoreMesh(core_axis_name="c", subcore_axis_name="s")
    f = functools.partial(
        pl.kernel,
        mesh=mesh,
        compiler_params=pltpu.CompilerParams(needs_layout_passes=False, use_tc_tiling_on_sc=False),
        out_type=jax.ShapeDtypeStruct((BATCH,), jnp.float32),
        scratch_types=[
            pltpu.VMEM((BPW,), jnp.int32),
            pltpu.VMEM((BPW,), jnp.int32),
            pltpu.VMEM((BPW, D), jnp.float32),
            pltpu.VMEM((BPW, D), jnp.float32),
            pltpu.VMEM((D,), jnp.float32),
            pltpu.VMEM((L,), jnp.float32),
            pltpu.VMEM((BPW,), jnp.float32),
            pltpu.SemaphoreType.DMA,
            pltpu.SemaphoreType.DMA,
        ],
    )(_sc_body)
    return f(uidx, iidx, utab, itab, w_flat, b_pad)


def kernel(user_indices, item_indices, user_table, item_table, W, b):
    w_flat = W.reshape(D)
    b_pad = jnp.pad(b.astype(jnp.float32), (0, L - b.shape[0]))
    out = _gmf_sc(user_indices.astype(jnp.int32), item_indices.astype(jnp.int32),
                  user_table, item_table, w_flat, b_pad)
    return out.reshape(BATCH, 1)
